# baseline (speedup 1.0000x reference)
import functools

import jax
import jax.numpy as jnp
from jax import lax
from jax.experimental import pallas as pl
from jax.experimental.pallas import tpu as pltpu
from jax.experimental.pallas import tpu_sc as plsc

B, N, C = 2, 2048, 1024
T = B * N
E, TOPK = 8, 2
INTER = int(C * 8 // 3)
IPAD = 2816
CAP = int(1.25 * T / E)
H = C // 2
NC_SC, NS_SC = 2, 16
NW = NC_SC * NS_SC
ZERO_ROW = E * CAP
TB = 1024


def _gate_body(x_ref, wg1_ref, bg1_ref, wg2_ref, p_ref):
    xf = x_ref[...]
    mean = jnp.mean(xf, axis=-1, keepdims=True)
    var = jnp.sum((xf - mean) ** 2, axis=-1, keepdims=True) / (C - 1)
    std = jnp.sqrt(var)
    mn = jnp.min(xf, axis=-1, keepdims=True)
    mx = jnp.max(xf, axis=-1, keepdims=True)
    l2 = jnp.sqrt(jnp.sum(xf * xf, axis=-1, keepdims=True))
    sp = jnp.mean((jnp.abs(xf) < 1e-06).astype(jnp.float32), axis=-1,
                  keepdims=True)
    gate_in = jnp.concatenate([xf, mean, std, mn, mx, l2, sp], axis=-1)
    h = jnp.dot(gate_in, wg1_ref[...], preferred_element_type=jnp.float32)
    h = h + bg1_ref[...]
    h = h * 0.5 * (1.0 + lax.erf(h * (2.0 ** -0.5)))
    logits = jnp.dot(h, wg2_ref[...], preferred_element_type=jnp.float32)
    lm = jnp.max(logits, axis=-1, keepdims=True)
    ex = jnp.exp(logits - lm)
    probs = ex / jnp.sum(ex, axis=-1, keepdims=True)
    io = lax.broadcasted_iota(jnp.int32, (TB, E), 1)
    m0 = jnp.max(probs, axis=-1, keepdims=True)
    i0 = jnp.min(jnp.where(probs == m0, io, E), axis=-1, keepdims=True)
    masked = jnp.where(io == i0, -jnp.inf, probs)
    m1 = jnp.max(masked, axis=-1, keepdims=True)
    i1 = jnp.min(jnp.where((masked == m1) & (io != i0), io, E), axis=-1,
                 keepdims=True)
    denom = jnp.maximum(jnp.abs(m0) + jnp.abs(m1), 1e-12)
    w0 = m0 / denom
    w1 = m1 / denom
    p_ref[...] = jnp.where(io == i0, w0, jnp.where(io == i1, w1, -1.0))


def _gate(xf, Wg1, bg1, Wg2):
    nblk = T // TB
    return pl.pallas_call(
        _gate_body,
        grid=(nblk,),
        in_specs=[
            pl.BlockSpec((TB, C), lambda i: (i, 0)),
            pl.BlockSpec((C + 6, H), lambda i: (0, 0)),
            pl.BlockSpec((H,), lambda i: (0,)),
            pl.BlockSpec((H, E), lambda i: (0, 0)),
        ],
        out_specs=pl.BlockSpec((TB, E), lambda i: (i, 0)),
        out_shape=jax.ShapeDtypeStruct((T, E), jnp.float32),
    )(xf, Wg1, bg1, Wg2)


def _excl_cumsum(x):
    s = x
    sh = 1
    while sh < T:
        s = s + jnp.concatenate(
            [jnp.zeros((E, sh), s.dtype), s[:, :-sh]], axis=1)
        sh *= 2
    return s - x


def _route_body(p_ref, tok_ref, wsel_ref, krank_ref):
    p = p_ref[...]
    wc = p.T
    lo = jnp.zeros((E, 1), jnp.int32)
    hi = jnp.full((E, 1), 0x3F800001, jnp.int32)
    for _ in range(31):
        mid = (lo + hi) // 2
        v = lax.bitcast_convert_type(mid, jnp.float32)
        cnt = jnp.sum((wc >= v).astype(jnp.int32), axis=1, keepdims=True)
        ge = cnt >= CAP
        lo = jnp.where(ge, mid, lo)
        hi = jnp.where(ge, hi, mid)
    vstar = lax.bitcast_convert_type(lo, jnp.float32)
    n_assigned = jnp.sum((wc >= 0.0).astype(jnp.int32), axis=1, keepdims=True)
    small = n_assigned < CAP
    strict_i = jnp.where(small, (wc >= 0.0).astype(jnp.int32),
                         (wc > vstar).astype(jnp.int32))
    tie_i = jnp.where(small, (wc == -1.0).astype(jnp.int32),
                      (wc == vstar).astype(jnp.int32))
    n_strict = jnp.sum(strict_i, axis=1, keepdims=True)
    tierank = _excl_cumsum(tie_i)
    fill_i = tie_i * (tierank < (CAP - n_strict)).astype(jnp.int32)
    sel_i = strict_i + fill_i
    rank = _excl_cumsum(sel_i)

    tcol = lax.broadcasted_iota(jnp.int32, (T, 1), 0).astype(jnp.float32)
    pio = lax.broadcasted_iota(jnp.int32, (CAP, T), 0)
    for e in range(E):
        re = ((rank[e:e + 1, :] == pio).astype(jnp.float32)
              * sel_i[e:e + 1, :].astype(jnp.float32))
        rhs = jnp.concatenate([tcol, p[:, e:e + 1]], axis=1)
        out_e = jnp.dot(re, rhs, preferred_element_type=jnp.float32)
        tok_ref[e, :] = out_e[:, 0].astype(jnp.int32)
        wsel_ref[e, :] = out_e[:, 1]

    kept = (sel_i * (wc >= 0.0).astype(jnp.int32)) > 0
    krank = jnp.where(kept, rank, -1)
    krank_ref[...] = krank.T


def _route(p):
    return pl.pallas_call(
        _route_body,
        out_shape=(
            jax.ShapeDtypeStruct((E, CAP), jnp.int32),
            jax.ShapeDtypeStruct((E, CAP), jnp.float32),
            jax.ShapeDtypeStruct((T, E), jnp.int32),
        ),
    )(p)


def _sc_gather(xf, tok_flat):
    rpw = (E * CAP) // NW
    nch, ch = 4, 40
    mesh = plsc.VectorSubcoreMesh(core_axis_name="c", subcore_axis_name="s")

    @functools.partial(
        pl.kernel, mesh=mesh,
        out_type=jax.ShapeDtypeStruct((E * CAP, C), jnp.float32),
        scratch_types=[
            pltpu.VMEM((rpw,), jnp.int32),
            pltpu.VMEM((ch, C), jnp.float32),
            pltpu.VMEM((ch, C), jnp.float32),
            pltpu.SemaphoreType.DMA,
            pltpu.SemaphoreType.DMA,
            pltpu.SemaphoreType.DMA,
            pltpu.SemaphoreType.DMA,
        ],
    )
    def k(xf_hbm, tok_hbm, out_hbm, idx_v, r0, r1, g0, g1, s0, s1):
        wid = lax.axis_index("s") * NC_SC + lax.axis_index("c")
        base = wid * rpw
        pltpu.sync_copy(tok_hbm.at[pl.ds(base, rpw)], idx_v)
        bufs, gsems, ssems = (r0, r1), (g0, g1), (s0, s1)
        gh = [None] * nch
        sh = [None] * nch
        gh[0] = pltpu.async_copy(xf_hbm.at[idx_v.at[pl.ds(0, ch)]], r0, g0)
        for c in range(nch):
            b = c & 1
            gh[c].wait()
            if c + 1 < nch:
                if c - 1 >= 0:
                    sh[c - 1].wait()
                gh[c + 1] = pltpu.async_copy(
                    xf_hbm.at[idx_v.at[pl.ds((c + 1) * ch, ch)]],
                    bufs[1 - b], gsems[1 - b])
            sh[c] = pltpu.async_copy(
                bufs[b], out_hbm.at[pl.ds(base + c * ch, ch)], ssems[b])
        sh[nch - 2].wait()
        sh[nch - 1].wait()

    return k(xf, tok_flat)


_KC = 8
_KB = C // _KC
_SPE = _KC + 4
_NTB = 8
_TBC = T // _NTB
_CH4 = C // 4


def _ffnc_body(xs_ref, w1_ref, w2_ref, ws_ref, kr_ref, o_ref,
               u_acc, g_s, eo_s):
    i = pl.program_id(0)
    e = jnp.minimum(i // _SPE, E - 1)
    s = i - e * _SPE

    @pl.when(jnp.logical_and(i < E * _SPE, s < _KC))
    def _():
        xsb = xs_ref[0].astype(jnp.bfloat16)
        w1b = w1_ref[0].astype(jnp.bfloat16)
        part = jnp.dot(xsb, w1b, preferred_element_type=jnp.float32)

        @pl.when(s == 0)
        def _():
            u_acc[...] = part

        @pl.when(s > 0)
        def _():
            u_acc[...] = u_acc[...] + part

        @pl.when(s == _KC - 1)
        def _():
            u = u_acc[...]
            ua = u[:, :INTER]
            ub = u[:, INTER:]
            g_s[...] = ((ua * jax.nn.sigmoid(ua)) * ub).astype(jnp.bfloat16)

    @pl.when(jnp.logical_and(i < E * _SPE, s >= _KC))
    def _():
        half = s - _KC
        w2b = w2_ref[0].astype(jnp.bfloat16)
        part = jnp.dot(g_s[...], w2b, preferred_element_type=jnp.float32)
        eo_s[pl.ds(e * CAP, CAP), pl.ds(half * _CH4, _CH4)] = (
            (part * ws_ref[0]).astype(jnp.bfloat16))

    @pl.when(i >= E * _SPE)
    def _():
        kr = kr_ref[...]
        pio = lax.broadcasted_iota(jnp.int32, (_TBC, CAP), 1)
        sm = jnp.concatenate(
            [(pio == kr[:, q:q + 1]).astype(jnp.bfloat16) for q in range(E)],
            axis=1)
        o_ref[...] = jnp.dot(sm, eo_s[...],
                             preferred_element_type=jnp.float32)


def _ffnc(xs, w1, w2, ws, krank):
    ee = lambda i: jnp.minimum(i // _SPE, E - 1)
    tb = lambda i: jnp.maximum(i - E * _SPE, 0)
    return pl.pallas_call(
        _ffnc_body,
        grid=(E * _SPE + _NTB,),
        in_specs=[
            pl.BlockSpec((1, CAP, _KB),
                         lambda i: (ee(i), 0,
                                    jnp.minimum(i - ee(i) * _SPE, _KC - 1))),
            pl.BlockSpec((1, _KB, 2 * INTER),
                         lambda i: (ee(i),
                                    jnp.minimum(i - ee(i) * _SPE, _KC - 1),
                                    0)),
            pl.BlockSpec((1, INTER, _CH4),
                         lambda i: (ee(i), 0,
                                    jnp.clip(i - ee(i) * _SPE - _KC, 0, 3))),
            pl.BlockSpec((1, CAP, 1), lambda i: (ee(i), 0, 0)),
            pl.BlockSpec((_TBC, E), lambda i: (tb(i), 0)),
        ],
        out_specs=pl.BlockSpec((_TBC, C), lambda i: (tb(i), 0)),
        out_shape=jax.ShapeDtypeStruct((T, C), jnp.float32),
        scratch_shapes=[
            pltpu.VMEM((CAP, 2 * INTER), jnp.float32),
            pltpu.VMEM((CAP, INTER), jnp.bfloat16),
            pltpu.VMEM((E * CAP, C), jnp.bfloat16),
        ],
        compiler_params=pltpu.CompilerParams(
            dimension_semantics=("arbitrary",)),
    )(xs, w1, w2, ws, krank)


def kernel(x, t, snr_threshold, Wg1, bg1, Wg2, W1, W2):
    xf = x.reshape(-1, C)
    p = _gate(xf, Wg1, bg1, Wg2)
    tok, wsel, krank = _route(p)
    xs = _sc_gather(xf, tok.reshape(-1))
    out = _ffnc(xs.reshape(E, CAP, C), W1, W2,
                wsel.reshape(E, CAP, 1), krank)
    return out.reshape(x.shape), jnp.float32(0.0)


# speedup vs baseline: 1.2555x; 1.2555x over previous
import functools

import jax
import jax.numpy as jnp
from jax import lax
from jax.experimental import pallas as pl
from jax.experimental.pallas import tpu as pltpu
from jax.experimental.pallas import tpu_sc as plsc

B, N, C = 2, 2048, 1024
T = B * N
E, TOPK = 8, 2
INTER = int(C * 8 // 3)
IPAD = 2816
CAP = int(1.25 * T / E)
H = C // 2
NC_SC, NS_SC = 2, 16
NW = NC_SC * NS_SC
ZERO_ROW = E * CAP
TB = 1024


def _gate_body(x_ref, wg1_ref, bg1_ref, wg2_ref, p_ref):
    xf = x_ref[...]
    mean = jnp.mean(xf, axis=-1, keepdims=True)
    var = jnp.sum((xf - mean) ** 2, axis=-1, keepdims=True) / (C - 1)
    std = jnp.sqrt(var)
    mn = jnp.min(xf, axis=-1, keepdims=True)
    mx = jnp.max(xf, axis=-1, keepdims=True)
    l2 = jnp.sqrt(jnp.sum(xf * xf, axis=-1, keepdims=True))
    sp = jnp.mean((jnp.abs(xf) < 1e-06).astype(jnp.float32), axis=-1,
                  keepdims=True)
    gate_in = jnp.concatenate([xf, mean, std, mn, mx, l2, sp], axis=-1)
    h = jnp.dot(gate_in, wg1_ref[...], preferred_element_type=jnp.float32)
    h = h + bg1_ref[...]
    h = h * 0.5 * (1.0 + lax.erf(h * (2.0 ** -0.5)))
    logits = jnp.dot(h, wg2_ref[...], preferred_element_type=jnp.float32)
    lm = jnp.max(logits, axis=-1, keepdims=True)
    ex = jnp.exp(logits - lm)
    probs = ex / jnp.sum(ex, axis=-1, keepdims=True)
    io = lax.broadcasted_iota(jnp.int32, (TB, E), 1)
    m0 = jnp.max(probs, axis=-1, keepdims=True)
    i0 = jnp.min(jnp.where(probs == m0, io, E), axis=-1, keepdims=True)
    masked = jnp.where(io == i0, -jnp.inf, probs)
    m1 = jnp.max(masked, axis=-1, keepdims=True)
    i1 = jnp.min(jnp.where((masked == m1) & (io != i0), io, E), axis=-1,
                 keepdims=True)
    denom = jnp.maximum(jnp.abs(m0) + jnp.abs(m1), 1e-12)
    w0 = m0 / denom
    w1 = m1 / denom
    p_ref[...] = jnp.where(io == i0, w0, jnp.where(io == i1, w1, -1.0))


def _gate(xf, Wg1, bg1, Wg2):
    nblk = T // TB
    return pl.pallas_call(
        _gate_body,
        grid=(nblk,),
        in_specs=[
            pl.BlockSpec((TB, C), lambda i: (i, 0)),
            pl.BlockSpec((C + 6, H), lambda i: (0, 0)),
            pl.BlockSpec((H,), lambda i: (0,)),
            pl.BlockSpec((H, E), lambda i: (0, 0)),
        ],
        out_specs=pl.BlockSpec((TB, E), lambda i: (i, 0)),
        out_shape=jax.ShapeDtypeStruct((T, E), jnp.float32),
    )(xf, Wg1, bg1, Wg2)


def _excl_cumsum(x):
    s = x
    sh = 1
    while sh < T:
        s = s + jnp.concatenate(
            [jnp.zeros((E, sh), s.dtype), s[:, :-sh]], axis=1)
        sh *= 2
    return s - x


def _route_body(p_ref, tok_ref, wsel_ref, krank_ref):
    p = p_ref[...]
    wc = p.T
    lo = jnp.zeros((E, 1), jnp.int32)
    hi = jnp.full((E, 1), 0x3F800001, jnp.int32)
    for _ in range(31):
        mid = (lo + hi) // 2
        v = lax.bitcast_convert_type(mid, jnp.float32)
        cnt = jnp.sum((wc >= v).astype(jnp.int32), axis=1, keepdims=True)
        ge = cnt >= CAP
        lo = jnp.where(ge, mid, lo)
        hi = jnp.where(ge, hi, mid)
    vstar = lax.bitcast_convert_type(lo, jnp.float32)
    n_assigned = jnp.sum((wc >= 0.0).astype(jnp.int32), axis=1, keepdims=True)
    small = n_assigned < CAP
    strict_i = jnp.where(small, (wc >= 0.0).astype(jnp.int32),
                         (wc > vstar).astype(jnp.int32))
    tie_i = jnp.where(small, (wc == -1.0).astype(jnp.int32),
                      (wc == vstar).astype(jnp.int32))
    n_strict = jnp.sum(strict_i, axis=1, keepdims=True)
    tierank = _excl_cumsum(tie_i)
    fill_i = tie_i * (tierank < (CAP - n_strict)).astype(jnp.int32)
    sel_i = strict_i + fill_i
    rank = _excl_cumsum(sel_i)

    tcol = lax.broadcasted_iota(jnp.int32, (T, 1), 0).astype(jnp.float32)
    pio = lax.broadcasted_iota(jnp.int32, (CAP, T), 0)
    for e in range(E):
        re = ((rank[e:e + 1, :] == pio).astype(jnp.float32)
              * sel_i[e:e + 1, :].astype(jnp.float32))
        rhs = jnp.concatenate([tcol, p[:, e:e + 1]], axis=1)
        out_e = jnp.dot(re, rhs, preferred_element_type=jnp.float32)
        tok_ref[e, :] = out_e[:, 0].astype(jnp.int32)
        wsel_ref[e, :] = out_e[:, 1]

    kept = (sel_i * (wc >= 0.0).astype(jnp.int32)) > 0
    krank = jnp.where(kept, rank, -1)
    krank_ref[...] = krank.T


def _route(p):
    return pl.pallas_call(
        _route_body,
        out_shape=(
            jax.ShapeDtypeStruct((E, CAP), jnp.int32),
            jax.ShapeDtypeStruct((E, CAP), jnp.float32),
            jax.ShapeDtypeStruct((T, E), jnp.int32),
        ),
    )(p)


def _sc_gather(xf, tok_flat):
    rpw = (E * CAP) // NW
    nch, ch = 4, 40
    mesh = plsc.VectorSubcoreMesh(core_axis_name="c", subcore_axis_name="s")

    @functools.partial(
        pl.kernel, mesh=mesh,
        out_type=jax.ShapeDtypeStruct((E * CAP, C), jnp.float32),
        scratch_types=[
            pltpu.VMEM((rpw,), jnp.int32),
            pltpu.VMEM((ch, C), jnp.float32),
            pltpu.VMEM((ch, C), jnp.float32),
            pltpu.SemaphoreType.DMA,
            pltpu.SemaphoreType.DMA,
            pltpu.SemaphoreType.DMA,
            pltpu.SemaphoreType.DMA,
        ],
    )
    def k(xf_hbm, tok_hbm, out_hbm, idx_v, r0, r1, g0, g1, s0, s1):
        wid = lax.axis_index("s") * NC_SC + lax.axis_index("c")
        base = wid * rpw
        pltpu.sync_copy(tok_hbm.at[pl.ds(base, rpw)], idx_v)
        bufs, gsems, ssems = (r0, r1), (g0, g1), (s0, s1)
        gh = [None] * nch
        sh = [None] * nch
        gh[0] = pltpu.async_copy(xf_hbm.at[idx_v.at[pl.ds(0, ch)]], r0, g0)
        for c in range(nch):
            b = c & 1
            gh[c].wait()
            if c + 1 < nch:
                if c - 1 >= 0:
                    sh[c - 1].wait()
                gh[c + 1] = pltpu.async_copy(
                    xf_hbm.at[idx_v.at[pl.ds((c + 1) * ch, ch)]],
                    bufs[1 - b], gsems[1 - b])
            sh[c] = pltpu.async_copy(
                bufs[b], out_hbm.at[pl.ds(base + c * ch, ch)], ssems[b])
        sh[nch - 2].wait()
        sh[nch - 1].wait()

    return k(xf, tok_flat)


_KC = 2
_KB = C // _KC


def _ffn1_body(xs_ref, w1_ref, g_ref, u_acc):
    k = pl.program_id(1)
    xsb = xs_ref[0].astype(jnp.bfloat16)
    w1b = w1_ref[0].astype(jnp.bfloat16)
    part = jnp.dot(xsb, w1b, preferred_element_type=jnp.float32)

    @pl.when(k == 0)
    def _():
        u_acc[...] = part

    @pl.when(k > 0)
    def _():
        u_acc[...] = u_acc[...] + part

    @pl.when(k == _KC - 1)
    def _():
        u = u_acc[...]
        ua = u[:, :INTER]
        ub = u[:, INTER:]
        g_ref[0] = ((ua * jax.nn.sigmoid(ua)) * ub).astype(jnp.bfloat16)


def _ffn1(xs, w1):
    return pl.pallas_call(
        _ffn1_body,
        grid=(E, _KC),
        in_specs=[
            pl.BlockSpec((1, CAP, _KB), lambda e, k: (e, 0, k)),
            pl.BlockSpec((1, _KB, 2 * INTER), lambda e, k: (e, k, 0)),
        ],
        out_specs=pl.BlockSpec((1, CAP, INTER), lambda e, k: (e, 0, 0)),
        out_shape=jax.ShapeDtypeStruct((E, CAP, INTER), jnp.bfloat16),
        scratch_shapes=[pltpu.VMEM((CAP, 2 * INTER), jnp.float32)],
        compiler_params=pltpu.CompilerParams(
            dimension_semantics=("arbitrary", "arbitrary")),
    )(xs, w1)


def _ffn2_body(g_ref, w2_ref, ws_ref, eo_ref):
    w2b = w2_ref[0].astype(jnp.bfloat16)
    part = jnp.dot(g_ref[0], w2b, preferred_element_type=jnp.float32)
    eo_ref[...] = (part * ws_ref[0]).astype(jnp.bfloat16)


def _ffn2(g, w2, ws):
    return pl.pallas_call(
        _ffn2_body,
        grid=(E,),
        in_specs=[
            pl.BlockSpec((1, CAP, INTER), lambda e: (e, 0, 0)),
            pl.BlockSpec((1, INTER, C), lambda e: (e, 0, 0)),
            pl.BlockSpec((1, CAP, 1), lambda e: (e, 0, 0)),
        ],
        out_specs=pl.BlockSpec((CAP, C), lambda e: (e, 0)),
        out_shape=jax.ShapeDtypeStruct((E * CAP, C), jnp.bfloat16),
        compiler_params=pltpu.CompilerParams(
            dimension_semantics=("arbitrary",)),
    )(g, w2, ws)


def _combine_body(kr_ref, eo_ref, o_ref):
    kr = kr_ref[...]
    pio = lax.broadcasted_iota(jnp.int32, (TB, CAP), 1)
    s = jnp.concatenate(
        [(pio == kr[:, e:e + 1]).astype(jnp.bfloat16) for e in range(E)],
        axis=1)
    o_ref[...] = jnp.dot(s, eo_ref[...], preferred_element_type=jnp.float32)


def _combine(krank, eo):
    return pl.pallas_call(
        _combine_body,
        grid=(T // TB,),
        in_specs=[
            pl.BlockSpec((TB, E), lambda i: (i, 0)),
            pl.BlockSpec((E * CAP, C), lambda i: (0, 0)),
        ],
        out_specs=pl.BlockSpec((TB, C), lambda i: (i, 0)),
        out_shape=jax.ShapeDtypeStruct((T, C), jnp.float32),
        compiler_params=pltpu.CompilerParams(
            dimension_semantics=("arbitrary",)),
    )(krank, eo)


def kernel(x, t, snr_threshold, Wg1, bg1, Wg2, W1, W2):
    xf = x.reshape(-1, C)
    p = _gate(xf, Wg1, bg1, Wg2)
    tok, wsel, krank = _route(p)
    xs = _sc_gather(xf, tok.reshape(-1))
    g = _ffn1(xs.reshape(E, CAP, C), W1)
    eo = _ffn2(g, W2, wsel.reshape(E, CAP, 1))
    out = _combine(krank, eo)
    return out.reshape(x.shape), jnp.float32(0.0)


# speedup vs baseline: 1.2717x; 1.0129x over previous
import functools

import jax
import jax.numpy as jnp
from jax import lax
from jax.experimental import pallas as pl
from jax.experimental.pallas import tpu as pltpu
from jax.experimental.pallas import tpu_sc as plsc

B, N, C = 2, 2048, 1024
T = B * N
E, TOPK = 8, 2
INTER = int(C * 8 // 3)
IPAD = 2816
CAP = int(1.25 * T / E)
H = C // 2
NC_SC, NS_SC = 2, 16
NW = NC_SC * NS_SC
ZERO_ROW = E * CAP
TB = 1024


def _gate_body(x_ref, wg1_ref, bg1_ref, wg2_ref, p_ref):
    xf = x_ref[...]
    mean = jnp.mean(xf, axis=-1, keepdims=True)
    var = jnp.sum((xf - mean) ** 2, axis=-1, keepdims=True) / (C - 1)
    std = jnp.sqrt(var)
    mn = jnp.min(xf, axis=-1, keepdims=True)
    mx = jnp.max(xf, axis=-1, keepdims=True)
    l2 = jnp.sqrt(jnp.sum(xf * xf, axis=-1, keepdims=True))
    sp = jnp.mean((jnp.abs(xf) < 1e-06).astype(jnp.float32), axis=-1,
                  keepdims=True)
    gate_in = jnp.concatenate([xf, mean, std, mn, mx, l2, sp], axis=-1)
    h = jnp.dot(gate_in, wg1_ref[...], preferred_element_type=jnp.float32)
    h = h + bg1_ref[...]
    h = h * 0.5 * (1.0 + lax.erf(h * (2.0 ** -0.5)))
    logits = jnp.dot(h, wg2_ref[...], preferred_element_type=jnp.float32)
    lm = jnp.max(logits, axis=-1, keepdims=True)
    ex = jnp.exp(logits - lm)
    probs = ex / jnp.sum(ex, axis=-1, keepdims=True)
    io = lax.broadcasted_iota(jnp.int32, (TB, E), 1)
    m0 = jnp.max(probs, axis=-1, keepdims=True)
    i0 = jnp.min(jnp.where(probs == m0, io, E), axis=-1, keepdims=True)
    masked = jnp.where(io == i0, -jnp.inf, probs)
    m1 = jnp.max(masked, axis=-1, keepdims=True)
    i1 = jnp.min(jnp.where((masked == m1) & (io != i0), io, E), axis=-1,
                 keepdims=True)
    denom = jnp.maximum(jnp.abs(m0) + jnp.abs(m1), 1e-12)
    w0 = m0 / denom
    w1 = m1 / denom
    p_ref[...] = jnp.where(io == i0, w0, jnp.where(io == i1, w1, -1.0))


def _gate(xf, Wg1, bg1, Wg2):
    nblk = T // TB
    return pl.pallas_call(
        _gate_body,
        grid=(nblk,),
        in_specs=[
            pl.BlockSpec((TB, C), lambda i: (i, 0)),
            pl.BlockSpec((C + 6, H), lambda i: (0, 0)),
            pl.BlockSpec((H,), lambda i: (0,)),
            pl.BlockSpec((H, E), lambda i: (0, 0)),
        ],
        out_specs=pl.BlockSpec((TB, E), lambda i: (i, 0)),
        out_shape=jax.ShapeDtypeStruct((T, E), jnp.float32),
    )(xf, Wg1, bg1, Wg2)


def _excl_cumsum(x):
    s = x
    sh = 1
    while sh < T:
        s = s + jnp.concatenate(
            [jnp.zeros((E, sh), s.dtype), s[:, :-sh]], axis=1)
        sh *= 2
    return s - x


def _route_body(p_ref, tok_ref, wsel_ref, krank_ref):
    p = p_ref[...]
    wc = p.T
    lo = jnp.zeros((E, 1), jnp.int32)
    hi = jnp.full((E, 1), 0x3F800001, jnp.int32)
    for _ in range(31):
        mid = (lo + hi) // 2
        v = lax.bitcast_convert_type(mid, jnp.float32)
        cnt = jnp.sum((wc >= v).astype(jnp.int32), axis=1, keepdims=True)
        ge = cnt >= CAP
        lo = jnp.where(ge, mid, lo)
        hi = jnp.where(ge, hi, mid)
    vstar = lax.bitcast_convert_type(lo, jnp.float32)
    n_assigned = jnp.sum((wc >= 0.0).astype(jnp.int32), axis=1, keepdims=True)
    small = n_assigned < CAP
    strict_i = jnp.where(small, (wc >= 0.0).astype(jnp.int32),
                         (wc > vstar).astype(jnp.int32))
    tie_i = jnp.where(small, (wc == -1.0).astype(jnp.int32),
                      (wc == vstar).astype(jnp.int32))
    n_strict = jnp.sum(strict_i, axis=1, keepdims=True)
    tierank = _excl_cumsum(tie_i)
    fill_i = tie_i * (tierank < (CAP - n_strict)).astype(jnp.int32)
    sel_i = strict_i + fill_i
    rank = _excl_cumsum(sel_i)

    tcol = lax.broadcasted_iota(jnp.int32, (T, 1), 0).astype(jnp.float32)
    pio = lax.broadcasted_iota(jnp.int32, (CAP, T), 0)
    for e in range(E):
        re = ((rank[e:e + 1, :] == pio).astype(jnp.float32)
              * sel_i[e:e + 1, :].astype(jnp.float32))
        rhs = jnp.concatenate([tcol, p[:, e:e + 1]], axis=1)
        out_e = jnp.dot(re, rhs, preferred_element_type=jnp.float32)
        tok_ref[e, :] = out_e[:, 0].astype(jnp.int32)
        wsel_ref[e, :] = out_e[:, 1]

    kept = (sel_i * (wc >= 0.0).astype(jnp.int32)) > 0
    krank = jnp.where(kept, rank, -1)
    krank_ref[...] = krank.T


def _route(p):
    return pl.pallas_call(
        _route_body,
        out_shape=(
            jax.ShapeDtypeStruct((E, CAP), jnp.int32),
            jax.ShapeDtypeStruct((E, CAP), jnp.float32),
            jax.ShapeDtypeStruct((T, E), jnp.int32),
        ),
    )(p)


def _sc_gather(xf, tok_flat):
    rpw = (E * CAP) // NW
    nch, ch = 4, 40
    mesh = plsc.VectorSubcoreMesh(core_axis_name="c", subcore_axis_name="s")

    @functools.partial(
        pl.kernel, mesh=mesh,
        out_type=jax.ShapeDtypeStruct((E * CAP, C), jnp.float32),
        scratch_types=[
            pltpu.VMEM((rpw,), jnp.int32),
            pltpu.VMEM((ch, C), jnp.float32),
            pltpu.VMEM((ch, C), jnp.float32),
            pltpu.SemaphoreType.DMA,
            pltpu.SemaphoreType.DMA,
            pltpu.SemaphoreType.DMA,
            pltpu.SemaphoreType.DMA,
        ],
    )
    def k(xf_hbm, tok_hbm, out_hbm, idx_v, r0, r1, g0, g1, s0, s1):
        wid = lax.axis_index("s") * NC_SC + lax.axis_index("c")
        base = wid * rpw
        pltpu.sync_copy(tok_hbm.at[pl.ds(base, rpw)], idx_v)
        bufs, gsems, ssems = (r0, r1), (g0, g1), (s0, s1)
        gh = [None] * nch
        sh = [None] * nch
        gh[0] = pltpu.async_copy(xf_hbm.at[idx_v.at[pl.ds(0, ch)]], r0, g0)
        for c in range(nch):
            b = c & 1
            gh[c].wait()
            if c + 1 < nch:
                if c - 1 >= 0:
                    sh[c - 1].wait()
                gh[c + 1] = pltpu.async_copy(
                    xf_hbm.at[idx_v.at[pl.ds((c + 1) * ch, ch)]],
                    bufs[1 - b], gsems[1 - b])
            sh[c] = pltpu.async_copy(
                bufs[b], out_hbm.at[pl.ds(base + c * ch, ch)], ssems[b])
        sh[nch - 2].wait()
        sh[nch - 1].wait()

    return k(xf, tok_flat)


_KC = 2
_KB = C // _KC


def _ffn1_body(xs_ref, w1_ref, g_ref, u_acc):
    k = pl.program_id(1)
    xsb = xs_ref[0].astype(jnp.bfloat16)
    w1b = w1_ref[0].astype(jnp.bfloat16)
    part = jnp.dot(xsb, w1b, preferred_element_type=jnp.float32)

    @pl.when(k == 0)
    def _():
        u_acc[...] = part

    @pl.when(k > 0)
    def _():
        u_acc[...] = u_acc[...] + part

    @pl.when(k == _KC - 1)
    def _():
        u = u_acc[...]
        ua = u[:, :INTER]
        ub = u[:, INTER:]
        g_ref[0] = ((ua * jax.nn.sigmoid(ua)) * ub).astype(jnp.bfloat16)


def _ffn1(xs, w1):
    return pl.pallas_call(
        _ffn1_body,
        grid=(E, _KC),
        in_specs=[
            pl.BlockSpec((1, CAP, _KB), lambda e, k: (e, 0, k)),
            pl.BlockSpec((1, _KB, 2 * INTER), lambda e, k: (e, k, 0)),
        ],
        out_specs=pl.BlockSpec((1, CAP, INTER), lambda e, k: (e, 0, 0)),
        out_shape=jax.ShapeDtypeStruct((E, CAP, INTER), jnp.bfloat16),
        scratch_shapes=[pltpu.VMEM((CAP, 2 * INTER), jnp.float32)],
        compiler_params=pltpu.CompilerParams(
            dimension_semantics=("arbitrary", "arbitrary")),
    )(xs, w1)


_NTB = 8
_TBC = T // _NTB


def _ffn2c_body(g_ref, w2_ref, ws_ref, kr_ref, o_ref, eo_s):
    i = pl.program_id(0)

    @pl.when(i < E)
    def _():
        w2b = w2_ref[0].astype(jnp.bfloat16)
        part = jnp.dot(g_ref[0], w2b, preferred_element_type=jnp.float32)
        eo_s[pl.ds(i * CAP, CAP), :] = (part * ws_ref[0]).astype(jnp.bfloat16)

    @pl.when(i >= E)
    def _():
        kr = kr_ref[...]
        pio = lax.broadcasted_iota(jnp.int32, (_TBC, CAP), 1)
        sm = jnp.concatenate(
            [(pio == kr[:, q:q + 1]).astype(jnp.bfloat16) for q in range(E)],
            axis=1)
        o_ref[...] = jnp.dot(sm, eo_s[...],
                             preferred_element_type=jnp.float32)


def _ffn2c(g, w2, ws, krank):
    ec = lambda i: jnp.minimum(i, E - 1)
    tb = lambda i: jnp.maximum(i - E, 0)
    return pl.pallas_call(
        _ffn2c_body,
        grid=(E + _NTB,),
        in_specs=[
            pl.BlockSpec((1, CAP, INTER), lambda i: (ec(i), 0, 0)),
            pl.BlockSpec((1, INTER, C), lambda i: (ec(i), 0, 0)),
            pl.BlockSpec((1, CAP, 1), lambda i: (ec(i), 0, 0)),
            pl.BlockSpec((_TBC, E), lambda i: (tb(i), 0)),
        ],
        out_specs=pl.BlockSpec((_TBC, C), lambda i: (tb(i), 0)),
        out_shape=jax.ShapeDtypeStruct((T, C), jnp.float32),
        scratch_shapes=[pltpu.VMEM((E * CAP, C), jnp.bfloat16)],
        compiler_params=pltpu.CompilerParams(
            dimension_semantics=("arbitrary",)),
    )(g, w2, ws, krank)


def kernel(x, t, snr_threshold, Wg1, bg1, Wg2, W1, W2):
    xf = x.reshape(-1, C)
    p = _gate(xf, Wg1, bg1, Wg2)
    tok, wsel, krank = _route(p)
    xs = _sc_gather(xf, tok.reshape(-1))
    g = _ffn1(xs.reshape(E, CAP, C), W1)
    out = _ffn2c(g, W2, wsel.reshape(E, CAP, 1), krank)
    return out.reshape(x.shape), jnp.float32(0.0)
